# baseline (speedup 1.0000x reference)
import functools

import jax
import jax.numpy as jnp
from jax import lax
from jax.experimental import pallas as pl
from jax.experimental.pallas import tpu as pltpu
from jax.experimental.pallas import tpu_sc as plsc


def _dists(x_ref, w_ref, b_ref, cb_ref):
    xb = x_ref[0]
    z = jnp.dot(w_ref[...], xb, preferred_element_type=jnp.float32)
    z = z + b_ref[...][:, None]
    s = jnp.dot(cb_ref[...], z, preferred_element_type=jnp.float32)
    zz = jnp.sum(z * z, axis=0, keepdims=True)
    cc = jnp.sum(cb_ref[...] * cb_ref[...], axis=1, keepdims=True)
    return (zz - 2.0 * s) + cc


def _tc_body(x_ref, w_ref, b_ref, cb_ref, n_ref, i_ref):
    d2 = _dists(x_ref, w_ref, b_ref, cb_ref)
    n_ref[0] = jnp.min(d2, axis=0, keepdims=True)
    idx = jnp.argmin(d2, axis=0).astype(jnp.int32)
    i_ref[0] = idx.reshape(8, 128)


def _tc_stage(x, W, b, codebook):
    B, C_IN, T = x.shape
    C_OUT = W.shape[0]
    K = codebook.shape[0]
    return pl.pallas_call(
        _tc_body,
        grid=(B,),
        in_specs=[
            pl.BlockSpec((1, C_IN, T), lambda b_: (b_, 0, 0)),
            pl.BlockSpec((C_OUT, C_IN), lambda b_: (0, 0)),
            pl.BlockSpec((C_OUT,), lambda b_: (0,)),
            pl.BlockSpec((K, C_OUT), lambda b_: (0, 0)),
        ],
        out_specs=[
            pl.BlockSpec((1, 1, T), lambda b_: (b_, 0, 0)),
            pl.BlockSpec((1, 8, 128), lambda b_: (b_, 0, 0)),
        ],
        out_shape=[
            jax.ShapeDtypeStruct((B, 1, T), jnp.float32),
            jax.ShapeDtypeStruct((B, 8, 128), jnp.int32),
        ],
    )(x, W, b, codebook)


def _make_sc_gather(B, T, C_OUT, row0, n_workers=32, chunk=128):
    n_tokens = B * T
    t_per_w = n_tokens // n_workers
    n_chunks = t_per_w // chunk
    mesh = plsc.VectorSubcoreMesh(core_axis_name="c", subcore_axis_name="s",
                                  num_cores=1)

    @functools.partial(
        pl.kernel,
        mesh=mesh,
        compiler_params=pltpu.CompilerParams(use_tc_tiling_on_sc=False),
        out_type=jax.ShapeDtypeStruct((B, T, C_OUT), jnp.float32),
        scratch_types=[
            pltpu.VMEM((n_chunks, chunk), jnp.int32),
            pltpu.VMEM((t_per_w, C_OUT), jnp.float32),
            pltpu.SemaphoreType.DMA,
            pltpu.SemaphoreType.DMA,
        ],
    )
    def sc_gather(cb_hbm, idx_hbm, out_hbm, idx_v, rows_v, sem, osem):
        wid = lax.axis_index("s")
        base = wid * t_per_w
        pltpu.sync_copy(
            idx_hbm.at[pl.ds(row0 + wid * n_chunks, n_chunks)], idx_v)
        gathers = [
            pltpu.async_copy(cb_hbm.at[idx_v.at[c]],
                             rows_v.at[pl.ds(c * chunk, chunk)], sem)
            for c in range(n_chunks)
        ]
        writes = []
        for c, g in enumerate(gathers):
            g.wait()
            tok = base + c * chunk
            b_ = tok // T
            writes.append(
                pltpu.async_copy(rows_v.at[pl.ds(c * chunk, chunk)],
                                 out_hbm.at[b_, pl.ds(tok - b_ * T, chunk)],
                                 osem))
        for w_ in writes:
            w_.wait()

    return sc_gather


def kernel(x, W, b, codebook):
    B, C_IN, T = x.shape
    C_OUT = W.shape[0]
    K = codebook.shape[0]
    n, idx = _tc_stage(x, W, b, codebook)
    idx2d = idx.reshape(B * T // 128, 128)
    q_tm = _make_sc_gather(B, T, C_OUT, 0, n_workers=16)(codebook, idx2d)
    quantized = jnp.transpose(q_tm, (0, 2, 1))
    n = n.reshape(B, T)
    vq_norms = jnp.stack([n, n], axis=-1)
    return quantized, vq_norms


# speedup vs baseline: 1.0004x; 1.0004x over previous
import functools

import jax
import jax.numpy as jnp
from jax import lax
from jax.experimental import pallas as pl
from jax.experimental.pallas import tpu as pltpu
from jax.experimental.pallas import tpu_sc as plsc


def _dists(x_ref, w_ref, b_ref, cb_ref):
    xb = x_ref[0]
    z = jnp.dot(w_ref[...], xb, preferred_element_type=jnp.float32)
    z = z + b_ref[...][:, None]
    s = jnp.dot(cb_ref[...], z, preferred_element_type=jnp.float32)
    zz = jnp.sum(z * z, axis=0, keepdims=True)
    cc = jnp.sum(cb_ref[...] * cb_ref[...], axis=1, keepdims=True)
    return (zz - 2.0 * s) + cc


def _tc_body(x_ref, w_ref, b_ref, cb_ref, n_ref, i_ref):
    d2 = _dists(x_ref, w_ref, b_ref, cb_ref)
    n_ref[0] = jnp.min(d2, axis=0, keepdims=True)
    idx = jnp.argmin(d2, axis=0).astype(jnp.int32)
    i_ref[0] = idx.reshape(i_ref.shape[1], i_ref.shape[2])


def _tc_stage(x, W, b, codebook):
    B, C_IN, T = x.shape
    C_OUT = W.shape[0]
    K = codebook.shape[0]
    return pl.pallas_call(
        _tc_body,
        grid=(B,),
        in_specs=[
            pl.BlockSpec((1, C_IN, T), lambda b_: (b_, 0, 0)),
            pl.BlockSpec((C_OUT, C_IN), lambda b_: (0, 0)),
            pl.BlockSpec((C_OUT,), lambda b_: (0,)),
            pl.BlockSpec((K, C_OUT), lambda b_: (0, 0)),
        ],
        out_specs=[
            pl.BlockSpec((1, 1, T), lambda b_: (b_, 0, 0)),
            pl.BlockSpec((1, 8, 128), lambda b_: (b_, 0, 0)),
        ],
        out_shape=[
            jax.ShapeDtypeStruct((B, 1, T), jnp.float32),
            jax.ShapeDtypeStruct((B, 8, 128), jnp.int32),
        ],
    )(x, W, b, codebook)


def _make_sc_gather(B, T, C_OUT, n_workers=16, chunk=128):
    n_tokens = B * T
    t_per_w = n_tokens // n_workers
    n_chunks = t_per_w // chunk
    mesh = plsc.VectorSubcoreMesh(core_axis_name="c", subcore_axis_name="s",
                                  num_cores=1)

    @functools.partial(
        pl.kernel,
        mesh=mesh,
        compiler_params=pltpu.CompilerParams(use_tc_tiling_on_sc=False),
        out_type=jax.ShapeDtypeStruct((B, T, C_OUT), jnp.float32),
        scratch_types=[
            pltpu.VMEM((n_chunks, chunk), jnp.int32),
            pltpu.VMEM((t_per_w, C_OUT), jnp.float32),
            pltpu.SemaphoreType.DMA,
            pltpu.SemaphoreType.DMA,
        ],
    )
    def sc_gather(cb_hbm, idx_hbm, out_hbm, idx_v, rows_v, sem, osem):
        wid = lax.axis_index("s")
        base = wid * t_per_w
        pltpu.sync_copy(idx_hbm.at[pl.ds(wid * n_chunks, n_chunks)], idx_v)
        gathers = [
            pltpu.async_copy(cb_hbm.at[idx_v.at[c]],
                             rows_v.at[pl.ds(c * chunk, chunk)], sem)
            for c in range(n_chunks)
        ]
        writes = []
        for c, g in enumerate(gathers):
            g.wait()
            tok = base + c * chunk
            b_ = tok // T
            writes.append(
                pltpu.async_copy(rows_v.at[pl.ds(c * chunk, chunk)],
                                 out_hbm.at[b_, pl.ds(tok - b_ * T, chunk)],
                                 osem))
        for w_ in writes:
            w_.wait()

    return sc_gather


def kernel(x, W, b, codebook):
    B, C_IN, T = x.shape
    C_OUT = W.shape[0]
    K = codebook.shape[0]
    n, idx = _tc_stage(x, W, b, codebook)
    idx2d = idx.reshape(B * T // 128, 128)
    q_tm = _make_sc_gather(B, T, C_OUT)(codebook, idx2d)
    quantized = jnp.transpose(q_tm, (0, 2, 1))
    n = n.reshape(B, T)
    vq_norms = jnp.stack([n, n], axis=-1)
    return quantized, vq_norms


# speedup vs baseline: 1.0253x; 1.0249x over previous
import functools

import jax
import jax.numpy as jnp
from jax import lax
from jax.experimental import pallas as pl
from jax.experimental.pallas import tpu as pltpu
from jax.experimental.pallas import tpu_sc as plsc


def _dists(x_ref, w_ref, b_ref, cb_ref):
    xb = x_ref[0]
    z = jnp.dot(w_ref[...], xb, preferred_element_type=jnp.float32)
    z = z + b_ref[...][:, None]
    s = jnp.dot(cb_ref[...], z, preferred_element_type=jnp.float32)
    zz = jnp.sum(z * z, axis=0, keepdims=True)
    cc = jnp.sum(cb_ref[...] * cb_ref[...], axis=1, keepdims=True)
    return (zz - 2.0 * s) + cc


def _tc_body(x_ref, w_ref, b_ref, cb_ref, n_ref, i_ref):
    d2 = _dists(x_ref, w_ref, b_ref, cb_ref)
    m = jnp.min(d2, axis=0, keepdims=True)
    n_ref[0] = jnp.concatenate([m, m], axis=0)
    idx = jnp.argmin(d2, axis=0).astype(jnp.int32)
    i_ref[0] = idx.reshape(i_ref.shape[1], i_ref.shape[2])


def _tc_stage(x, W, b, codebook):
    B, C_IN, T = x.shape
    C_OUT = W.shape[0]
    K = codebook.shape[0]
    return pl.pallas_call(
        _tc_body,
        grid=(B,),
        in_specs=[
            pl.BlockSpec((1, C_IN, T), lambda b_: (b_, 0, 0)),
            pl.BlockSpec((C_OUT, C_IN), lambda b_: (0, 0)),
            pl.BlockSpec((C_OUT,), lambda b_: (0,)),
            pl.BlockSpec((K, C_OUT), lambda b_: (0, 0)),
        ],
        out_specs=[
            pl.BlockSpec((1, 2, T), lambda b_: (b_, 0, 0)),
            pl.BlockSpec((1, 8, 128), lambda b_: (b_, 0, 0)),
        ],
        out_shape=[
            jax.ShapeDtypeStruct((B, 2, T), jnp.float32),
            jax.ShapeDtypeStruct((B, 8, 128), jnp.int32),
        ],
    )(x, W, b, codebook)


def _make_sc_gather(B, T, C_OUT, n_workers=16, chunk=128):
    n_tokens = B * T
    t_per_w = n_tokens // n_workers
    n_chunks = t_per_w // chunk
    mesh = plsc.VectorSubcoreMesh(core_axis_name="c", subcore_axis_name="s",
                                  num_cores=1)

    @functools.partial(
        pl.kernel,
        mesh=mesh,
        compiler_params=pltpu.CompilerParams(use_tc_tiling_on_sc=False),
        out_type=jax.ShapeDtypeStruct((B, T, C_OUT), jnp.float32),
        scratch_types=[
            pltpu.VMEM((n_chunks, chunk), jnp.int32),
            pltpu.VMEM((t_per_w, C_OUT), jnp.float32),
            pltpu.SemaphoreType.DMA,
            pltpu.SemaphoreType.DMA,
        ],
    )
    def sc_gather(cb_hbm, idx_hbm, out_hbm, idx_v, rows_v, sem, osem):
        wid = lax.axis_index("s")
        base = wid * t_per_w
        pltpu.sync_copy(idx_hbm.at[pl.ds(wid * n_chunks, n_chunks)], idx_v)
        gathers = [
            pltpu.async_copy(cb_hbm.at[idx_v.at[c]],
                             rows_v.at[pl.ds(c * chunk, chunk)], sem)
            for c in range(n_chunks)
        ]
        writes = []
        for c, g in enumerate(gathers):
            g.wait()
            tok = base + c * chunk
            b_ = tok // T
            writes.append(
                pltpu.async_copy(rows_v.at[pl.ds(c * chunk, chunk)],
                                 out_hbm.at[b_, pl.ds(tok - b_ * T, chunk)],
                                 osem))
        for w_ in writes:
            w_.wait()

    return sc_gather


def kernel(x, W, b, codebook):
    B, C_IN, T = x.shape
    C_OUT = W.shape[0]
    K = codebook.shape[0]
    n, idx = _tc_stage(x, W, b, codebook)
    idx2d = idx.reshape(B * T // 128, 128)
    q_tm = _make_sc_gather(B, T, C_OUT)(codebook, idx2d)
    quantized = jnp.transpose(q_tm, (0, 2, 1))
    vq_norms = jnp.transpose(n, (0, 2, 1))
    return quantized, vq_norms


# speedup vs baseline: 1.0297x; 1.0043x over previous
import functools

import jax
import jax.numpy as jnp
from jax import lax
from jax.experimental import pallas as pl
from jax.experimental.pallas import tpu as pltpu
from jax.experimental.pallas import tpu_sc as plsc


def _dists(x_ref, w_ref, b_ref, cb_ref):
    xb = x_ref[0]
    z = jnp.dot(w_ref[...], xb, preferred_element_type=jnp.float32)
    z = z + b_ref[...][:, None]
    s = jnp.dot(cb_ref[...], z, preferred_element_type=jnp.float32)
    zz = jnp.sum(z * z, axis=0, keepdims=True)
    cc = jnp.sum(cb_ref[...] * cb_ref[...], axis=1, keepdims=True)
    return (zz - 2.0 * s) + cc


def _tc_body(x_ref, w_ref, b_ref, cb_ref, n_ref, i_ref):
    d2 = _dists(x_ref, w_ref, b_ref, cb_ref)
    m = jnp.min(d2, axis=0, keepdims=True)
    n_ref[0] = jnp.concatenate([m, m], axis=0)
    idx = jnp.argmin(d2, axis=0).astype(jnp.int32)
    i_ref[0] = idx.reshape(i_ref.shape[1], i_ref.shape[2])


def _tc_stage(x, W, b, codebook):
    B, C_IN, T = x.shape
    C_OUT = W.shape[0]
    K = codebook.shape[0]
    return pl.pallas_call(
        _tc_body,
        grid=(B,),
        in_specs=[
            pl.BlockSpec((1, C_IN, T), lambda b_: (b_, 0, 0)),
            pl.BlockSpec((C_OUT, C_IN), lambda b_: (0, 0)),
            pl.BlockSpec((C_OUT,), lambda b_: (0,)),
            pl.BlockSpec((K, C_OUT), lambda b_: (0, 0)),
        ],
        out_specs=[
            pl.BlockSpec((1, 2, T), lambda b_: (b_, 0, 0)),
            pl.BlockSpec((1, 8, 128), lambda b_: (b_, 0, 0)),
        ],
        out_shape=[
            jax.ShapeDtypeStruct((B, 2, T), jnp.float32),
            jax.ShapeDtypeStruct((B, 8, 128), jnp.int32),
        ],
    )(x, W, b, codebook)


def _make_sc_gather(B, T, C_OUT, n_workers=16, chunk=128):
    n_tokens = B * T
    t_per_w = n_tokens // n_workers
    n_chunks = t_per_w // chunk
    mesh = plsc.VectorSubcoreMesh(core_axis_name="c", subcore_axis_name="s",
                                  num_cores=1)

    @functools.partial(
        pl.kernel,
        mesh=mesh,
        compiler_params=pltpu.CompilerParams(use_tc_tiling_on_sc=False),
        out_type=jax.ShapeDtypeStruct((B, T, C_OUT), jnp.float32),
        scratch_types=[
            pltpu.VMEM((n_chunks, chunk), jnp.int32),
            pltpu.VMEM((t_per_w, C_OUT), jnp.float32),
            pltpu.SemaphoreType.DMA((n_chunks,)),
            pltpu.SemaphoreType.DMA,
        ],
    )
    def sc_gather(cb_hbm, idx_hbm, out_hbm, idx_v, rows_v, sem, osem):
        wid = lax.axis_index("s")
        base = wid * t_per_w
        pltpu.sync_copy(idx_hbm.at[pl.ds(wid * n_chunks, n_chunks)], idx_v)
        gathers = [
            pltpu.async_copy(cb_hbm.at[idx_v.at[c]],
                             rows_v.at[pl.ds(c * chunk, chunk)], sem.at[c])
            for c in range(n_chunks)
        ]
        writes = []
        for c, g in enumerate(gathers):
            g.wait()
            tok = base + c * chunk
            b_ = tok // T
            writes.append(
                pltpu.async_copy(rows_v.at[pl.ds(c * chunk, chunk)],
                                 out_hbm.at[b_, pl.ds(tok - b_ * T, chunk)],
                                 osem))
        for w_ in writes:
            w_.wait()

    return sc_gather


def kernel(x, W, b, codebook):
    B, C_IN, T = x.shape
    C_OUT = W.shape[0]
    K = codebook.shape[0]
    n, idx = _tc_stage(x, W, b, codebook)
    idx2d = idx.reshape(B * T // 128, 128)
    q_tm = _make_sc_gather(B, T, C_OUT)(codebook, idx2d)
    quantized = jnp.transpose(q_tm, (0, 2, 1))
    vq_norms = jnp.transpose(n, (0, 2, 1))
    return quantized, vq_norms
